# baseline (speedup 1.0000x reference)
import functools

import jax
import jax.numpy as jnp
from jax import lax
from jax.experimental import pallas as pl
from jax.experimental.pallas import tpu as pltpu
from jax.experimental.pallas import tpu_sc as plsc

_LANES = 16
_NC = 2
_NS = 16
_NW = _NC * _NS
_IDX_CHUNK = 128


def _tc_body(x_ref, t_ref, g_ref, bce_ref, *, c, ones_col):
    i = pl.program_id(0)
    x = x_ref[...]
    t = t_ref[...]
    col = lax.broadcasted_iota(jnp.int32, x.shape, 1)
    valid = col < c
    e = jnp.exp(-jnp.abs(x))
    bce = jnp.sum(jnp.where(
        valid, jnp.maximum(x, 0.0) - x * t + jnp.log1p(e), 0.0))
    s = 1.0 / (1.0 + e)
    pred = jnp.where(x >= 0, s, 1.0 - s)
    pred = jnp.where(valid, pred, 0.0)
    pred = jnp.where(col == ones_col, 1.0, pred)
    pred_h = pred.astype(jnp.bfloat16)
    g = lax.dot_general(pred_h, pred_h, (((0,), (0,)), ((), ())),
                        preferred_element_type=jnp.float32,
                        precision=lax.Precision.DEFAULT)

    @pl.when(i == 0)
    def _init():
        g_ref[...] = g
        bce_ref[...] = jnp.reshape(bce, (1, 1))

    @pl.when(i > 0)
    def _acc():
        g_ref[...] += g
        bce_ref[...] += jnp.reshape(bce, (1, 1))


def _tc_stage(x, t, cp, ones_col, row_tile):
    b, c = x.shape
    return pl.pallas_call(
        functools.partial(_tc_body, c=c, ones_col=ones_col),
        grid=(b // row_tile,),
        in_specs=[pl.BlockSpec((row_tile, cp), lambda i: (i, 0)),
                  pl.BlockSpec((row_tile, cp), lambda i: (i, 0))],
        out_specs=[pl.BlockSpec((cp, cp), lambda i: (0, 0)),
                   pl.BlockSpec((1, 1), lambda i: (0, 0))],
        out_shape=[jax.ShapeDtypeStruct((cp, cp), jnp.float32),
                   jax.ShapeDtypeStruct((1, 1), jnp.float32)],
    )(x, t)


def _make_sc_gather(cp, ones_col, impl_pw, dis_pw):
    mesh = plsc.VectorSubcoreMesh(core_axis_name="c", subcore_axis_name="s")

    @functools.partial(
        pl.kernel,
        mesh=mesh,
        out_type=jax.ShapeDtypeStruct((_NW, 3 * _LANES), jnp.float32),
        scratch_types=[
            pltpu.VMEM((impl_pw,), jnp.int32),
            pltpu.VMEM((impl_pw,), jnp.int32),
            pltpu.VMEM((dis_pw,), jnp.int32),
            pltpu.VMEM((dis_pw,), jnp.int32),
            pltpu.VMEM((impl_pw,), jnp.int32),
            pltpu.VMEM((impl_pw,), jnp.int32),
            pltpu.VMEM((dis_pw,), jnp.int32),
            pltpu.VMEM((impl_pw,), jnp.float32),
            pltpu.VMEM((impl_pw,), jnp.float32),
            pltpu.VMEM((dis_pw,), jnp.float32),
            pltpu.VMEM((3 * _LANES,), jnp.float32),
            pltpu.SemaphoreType.DMA,
        ],
    )
    def sc_gather(gflat_hbm, il_hbm, ir_hbm, dl_hbm, dr_hbm, out_hbm,
                  il_v, ir_v, dl_v, dr_v, gi_v, ci_v, di_v,
                  vg_v, vc_v, vd_v, part_v, sem):
        wid = lax.axis_index("s") * _NC + lax.axis_index("c")
        ib = wid * impl_pw
        db = wid * dis_pw
        pltpu.sync_copy(il_hbm.at[pl.ds(ib, impl_pw)], il_v)
        pltpu.sync_copy(ir_hbm.at[pl.ds(ib, impl_pw)], ir_v)
        pltpu.sync_copy(dl_hbm.at[pl.ds(db, dis_pw)], dl_v)
        pltpu.sync_copy(dr_hbm.at[pl.ds(db, dis_pw)], dr_v)
        for i in range(impl_pw // _LANES):
            sl = pl.ds(i * _LANES, _LANES)
            l = il_v[sl]
            gi_v[sl] = l * cp + ir_v[sl]
            ci_v[sl] = l * cp + ones_col
        for i in range(dis_pw // _LANES):
            sl = pl.ds(i * _LANES, _LANES)
            di_v[sl] = dl_v[sl] * cp + dr_v[sl]
        for j in range(impl_pw // _IDX_CHUNK):
            sl = pl.ds(j * _IDX_CHUNK, _IDX_CHUNK)
            pltpu.async_copy(gflat_hbm.at[gi_v.at[sl]], vg_v.at[sl], sem).wait()
            pltpu.async_copy(gflat_hbm.at[ci_v.at[sl]], vc_v.at[sl], sem).wait()
        for j in range(dis_pw // _IDX_CHUNK):
            sl = pl.ds(j * _IDX_CHUNK, _IDX_CHUNK)
            pltpu.async_copy(gflat_hbm.at[di_v.at[sl]], vd_v.at[sl], sem).wait()
        accg = vg_v[pl.ds(0, _LANES)]
        accc = vc_v[pl.ds(0, _LANES)]
        for i in range(1, impl_pw // _LANES):
            sl = pl.ds(i * _LANES, _LANES)
            accg = accg + vg_v[sl]
            accc = accc + vc_v[sl]
        accd = vd_v[pl.ds(0, _LANES)]
        for i in range(1, dis_pw // _LANES):
            accd = accd + vd_v[pl.ds(i * _LANES, _LANES)]
        part_v[pl.ds(0, _LANES)] = accg
        part_v[pl.ds(_LANES, _LANES)] = accc
        part_v[pl.ds(2 * _LANES, _LANES)] = accd
        pltpu.sync_copy(part_v, out_hbm.at[wid])

    return sc_gather


def _ceil_to(n, m):
    return -(-n // m) * m


def kernel(input, target, impl_l, impl_r, dis_l, dis_r):
    b, c = input.shape
    cp = _ceil_to(c + 2, 128)
    ones_col = cp - 2
    zpad = cp - 1

    n_impl = impl_l.shape[0]
    n_dis = dis_l.shape[0]
    impl_pw = _ceil_to(-(-n_impl // _NW), _IDX_CHUNK)
    dis_pw = _ceil_to(-(-n_dis // _NW), _IDX_CHUNK)
    il = jnp.pad(impl_l, (0, _NW * impl_pw - n_impl), constant_values=zpad)
    ir = jnp.pad(impl_r, (0, _NW * impl_pw - n_impl), constant_values=zpad)
    dl = jnp.pad(dis_l, (0, _NW * dis_pw - n_dis), constant_values=zpad)
    dr = jnp.pad(dis_r, (0, _NW * dis_pw - n_dis), constant_values=zpad)

    g, bce = _tc_stage(input, target, cp, ones_col, row_tile=256)
    gflat = jnp.reshape(g, (cp * cp,))
    parts = _make_sc_gather(cp, ones_col, impl_pw, dis_pw)(gflat, il, ir, dl, dr)

    sums = jnp.sum(jnp.reshape(parts, (_NW, 3, _LANES)), axis=(0, 2))
    base_loss = bce[0, 0] / (b * c)
    implication_loss = (sums[1] - sums[0]) / b
    disjointness_loss = sums[2] / b
    loss = base_loss + 0.1 * implication_loss
    total = loss + 100.0 * disjointness_loss
    return (total, base_loss, implication_loss, disjointness_loss)


# speedup vs baseline: 1.0098x; 1.0098x over previous
import functools

import jax
import jax.numpy as jnp
from jax import lax
from jax.experimental import pallas as pl
from jax.experimental.pallas import tpu as pltpu
from jax.experimental.pallas import tpu_sc as plsc

_LANES = 16
_NC = 2
_NS = 16
_NW = _NC * _NS
_IDX_CHUNK = 128


def _tc_body(x_ref, t_ref, g_ref, bce_ref, *, c, ones_col):
    i = pl.program_id(0)
    x = x_ref[...]
    t = t_ref[...]
    col = lax.broadcasted_iota(jnp.int32, x.shape, 1)
    valid = col < c
    e = jnp.exp(-jnp.abs(x))
    bce = jnp.sum(jnp.where(
        valid, jnp.maximum(x, 0.0) - x * t + jnp.log1p(e), 0.0))
    s = 1.0 / (1.0 + e)
    pred = jnp.where(x >= 0, s, 1.0 - s)
    pred = jnp.where(valid, pred, 0.0)
    pred = jnp.where(col == ones_col, 1.0, pred)
    g = lax.dot_general(pred, pred, (((0,), (0,)), ((), ())),
                        preferred_element_type=jnp.float32,
                        precision=lax.Precision.DEFAULT)

    @pl.when(i == 0)
    def _init():
        g_ref[...] = g
        bce_ref[...] = jnp.reshape(bce, (1, 1))

    @pl.when(i > 0)
    def _acc():
        g_ref[...] += g
        bce_ref[...] += jnp.reshape(bce, (1, 1))


def _tc_stage(x, t, cp, ones_col, row_tile):
    b, c = x.shape
    return pl.pallas_call(
        functools.partial(_tc_body, c=c, ones_col=ones_col),
        grid=(b // row_tile,),
        in_specs=[pl.BlockSpec((row_tile, cp), lambda i: (i, 0)),
                  pl.BlockSpec((row_tile, cp), lambda i: (i, 0))],
        out_specs=[pl.BlockSpec((cp, cp), lambda i: (0, 0)),
                   pl.BlockSpec((1, 1), lambda i: (0, 0))],
        out_shape=[jax.ShapeDtypeStruct((cp, cp), jnp.float32),
                   jax.ShapeDtypeStruct((1, 1), jnp.float32)],
    )(x, t)


def _make_sc_gather(cp, ones_col, impl_pw, dis_pw):
    mesh = plsc.VectorSubcoreMesh(core_axis_name="c", subcore_axis_name="s")

    @functools.partial(
        pl.kernel,
        mesh=mesh,
        out_type=jax.ShapeDtypeStruct((_NW, 3 * _LANES), jnp.float32),
        scratch_types=[
            pltpu.VMEM((impl_pw,), jnp.int32),
            pltpu.VMEM((impl_pw,), jnp.int32),
            pltpu.VMEM((dis_pw,), jnp.int32),
            pltpu.VMEM((dis_pw,), jnp.int32),
            pltpu.VMEM((impl_pw,), jnp.int32),
            pltpu.VMEM((impl_pw,), jnp.int32),
            pltpu.VMEM((dis_pw,), jnp.int32),
            pltpu.VMEM((impl_pw,), jnp.float32),
            pltpu.VMEM((impl_pw,), jnp.float32),
            pltpu.VMEM((dis_pw,), jnp.float32),
            pltpu.VMEM((3 * _LANES,), jnp.float32),
            pltpu.SemaphoreType.DMA,
        ],
    )
    def sc_gather(gflat_hbm, il_hbm, ir_hbm, dl_hbm, dr_hbm, out_hbm,
                  il_v, ir_v, dl_v, dr_v, gi_v, ci_v, di_v,
                  vg_v, vc_v, vd_v, part_v, sem):
        wid = lax.axis_index("s") * _NC + lax.axis_index("c")
        ib = wid * impl_pw
        db = wid * dis_pw
        pltpu.sync_copy(il_hbm.at[pl.ds(ib, impl_pw)], il_v)
        pltpu.sync_copy(ir_hbm.at[pl.ds(ib, impl_pw)], ir_v)
        pltpu.sync_copy(dl_hbm.at[pl.ds(db, dis_pw)], dl_v)
        pltpu.sync_copy(dr_hbm.at[pl.ds(db, dis_pw)], dr_v)
        for i in range(impl_pw // _LANES):
            sl = pl.ds(i * _LANES, _LANES)
            l = il_v[sl]
            gi_v[sl] = l * cp + ir_v[sl]
            ci_v[sl] = l * cp + ones_col
        for i in range(dis_pw // _LANES):
            sl = pl.ds(i * _LANES, _LANES)
            di_v[sl] = dl_v[sl] * cp + dr_v[sl]
        for j in range(impl_pw // _IDX_CHUNK):
            sl = pl.ds(j * _IDX_CHUNK, _IDX_CHUNK)
            pltpu.async_copy(gflat_hbm.at[gi_v.at[sl]], vg_v.at[sl], sem).wait()
            pltpu.async_copy(gflat_hbm.at[ci_v.at[sl]], vc_v.at[sl], sem).wait()
        for j in range(dis_pw // _IDX_CHUNK):
            sl = pl.ds(j * _IDX_CHUNK, _IDX_CHUNK)
            pltpu.async_copy(gflat_hbm.at[di_v.at[sl]], vd_v.at[sl], sem).wait()
        accg = vg_v[pl.ds(0, _LANES)]
        accc = vc_v[pl.ds(0, _LANES)]
        for i in range(1, impl_pw // _LANES):
            sl = pl.ds(i * _LANES, _LANES)
            accg = accg + vg_v[sl]
            accc = accc + vc_v[sl]
        accd = vd_v[pl.ds(0, _LANES)]
        for i in range(1, dis_pw // _LANES):
            accd = accd + vd_v[pl.ds(i * _LANES, _LANES)]
        part_v[pl.ds(0, _LANES)] = accg
        part_v[pl.ds(_LANES, _LANES)] = accc
        part_v[pl.ds(2 * _LANES, _LANES)] = accd
        pltpu.sync_copy(part_v, out_hbm.at[wid])

    return sc_gather


def _ceil_to(n, m):
    return -(-n // m) * m


def kernel(input, target, impl_l, impl_r, dis_l, dis_r):
    b, c = input.shape
    cp = _ceil_to(c + 2, 128)
    ones_col = cp - 2
    zpad = cp - 1

    n_impl = impl_l.shape[0]
    n_dis = dis_l.shape[0]
    impl_pw = _ceil_to(-(-n_impl // _NW), _IDX_CHUNK)
    dis_pw = _ceil_to(-(-n_dis // _NW), _IDX_CHUNK)
    il = jnp.pad(impl_l, (0, _NW * impl_pw - n_impl), constant_values=zpad)
    ir = jnp.pad(impl_r, (0, _NW * impl_pw - n_impl), constant_values=zpad)
    dl = jnp.pad(dis_l, (0, _NW * dis_pw - n_dis), constant_values=zpad)
    dr = jnp.pad(dis_r, (0, _NW * dis_pw - n_dis), constant_values=zpad)

    g, bce = _tc_stage(input, target, cp, ones_col, row_tile=256)
    gflat = jnp.reshape(g, (cp * cp,))
    parts = _make_sc_gather(cp, ones_col, impl_pw, dis_pw)(gflat, il, ir, dl, dr)

    sums = jnp.sum(jnp.reshape(parts, (_NW, 3, _LANES)), axis=(0, 2))
    base_loss = bce[0, 0] / (b * c)
    implication_loss = (sums[1] - sums[0]) / b
    disjointness_loss = sums[2] / b
    loss = base_loss + 0.1 * implication_loss
    total = loss + 100.0 * disjointness_loss
    return (total, base_loss, implication_loss, disjointness_loss)


# speedup vs baseline: 1.0533x; 1.0430x over previous
import functools

import jax
import jax.numpy as jnp
from jax import lax
from jax.experimental import pallas as pl
from jax.experimental.pallas import tpu as pltpu
from jax.experimental.pallas import tpu_sc as plsc

_LANES = 16
_NC = 2
_NS = 16
_NW = _NC * _NS
_IDX_CHUNK = 128


def _tc_body(x_ref, t_ref, g_ref, cs_ref, bce_ref):
    i = pl.program_id(0)
    x = x_ref[...]
    t = t_ref[...]
    e = jnp.exp(-jnp.abs(x))
    bce = jnp.sum(jnp.maximum(x, 0.0) - x * t + jnp.log1p(e))
    s = 1.0 / (1.0 + e)
    pred = jnp.where(x >= 0, s, 1.0 - s)
    g = lax.dot_general(pred, pred, (((0,), (0,)), ((), ())),
                        preferred_element_type=jnp.float32,
                        precision=lax.Precision.DEFAULT)
    cs = jnp.sum(pred, axis=0, keepdims=True)

    @pl.when(i == 0)
    def _init():
        g_ref[...] = g
        cs_ref[...] = cs
        bce_ref[...] = jnp.reshape(bce, (1, 1))

    @pl.when(i > 0)
    def _acc():
        g_ref[...] += g
        cs_ref[...] += cs
        bce_ref[...] += jnp.reshape(bce, (1, 1))


def _tc_stage(x, t, row_tile):
    b, c = x.shape
    return pl.pallas_call(
        _tc_body,
        grid=(b // row_tile,),
        in_specs=[pl.BlockSpec((row_tile, c), lambda i: (i, 0)),
                  pl.BlockSpec((row_tile, c), lambda i: (i, 0))],
        out_specs=[pl.BlockSpec((c, c), lambda i: (0, 0)),
                   pl.BlockSpec((1, c), lambda i: (0, 0)),
                   pl.BlockSpec((1, 1), lambda i: (0, 0))],
        out_shape=[jax.ShapeDtypeStruct((c, c), jnp.float32),
                   jax.ShapeDtypeStruct((1, c), jnp.float32),
                   jax.ShapeDtypeStruct((1, 1), jnp.float32)],
    )(x, t)


def _make_sc_gather(c, n_impl, n_dis, impl_pw, dis_pw):
    mesh = plsc.VectorSubcoreMesh(core_axis_name="c", subcore_axis_name="s")

    @functools.partial(
        pl.kernel,
        mesh=mesh,
        out_type=jax.ShapeDtypeStruct((_NW, 3 * _LANES), jnp.float32),
        scratch_types=[
            pltpu.VMEM((impl_pw,), jnp.int32),
            pltpu.VMEM((impl_pw,), jnp.int32),
            pltpu.VMEM((dis_pw,), jnp.int32),
            pltpu.VMEM((dis_pw,), jnp.int32),
            pltpu.VMEM((impl_pw,), jnp.int32),
            pltpu.VMEM((dis_pw,), jnp.int32),
            pltpu.VMEM((impl_pw,), jnp.float32),
            pltpu.VMEM((impl_pw,), jnp.float32),
            pltpu.VMEM((dis_pw,), jnp.float32),
            pltpu.VMEM((3 * _LANES,), jnp.float32),
            pltpu.SemaphoreType.DMA,
        ],
    )
    def sc_gather(gflat_hbm, cs_hbm, il_hbm, ir_hbm, dl_hbm, dr_hbm, out_hbm,
                  il_v, ir_v, dl_v, dr_v, gi_v, di_v,
                  vg_v, vc_v, vd_v, part_v, sem):
        wid = lax.axis_index("s") * _NC + lax.axis_index("c")
        ib = wid * impl_pw
        db = wid * dis_pw
        pltpu.sync_copy(il_hbm.at[pl.ds(ib, impl_pw)], il_v)
        pltpu.sync_copy(ir_hbm.at[pl.ds(ib, impl_pw)], ir_v)
        pltpu.sync_copy(dl_hbm.at[pl.ds(db, dis_pw)], dl_v)
        pltpu.sync_copy(dr_hbm.at[pl.ds(db, dis_pw)], dr_v)
        for i in range(impl_pw // _LANES):
            sl = pl.ds(i * _LANES, _LANES)
            gi_v[sl] = il_v[sl] * c + ir_v[sl]
        for i in range(dis_pw // _LANES):
            sl = pl.ds(i * _LANES, _LANES)
            di_v[sl] = dl_v[sl] * c + dr_v[sl]
        copies = []
        for j in range(impl_pw // _IDX_CHUNK):
            sl = pl.ds(j * _IDX_CHUNK, _IDX_CHUNK)
            copies.append(pltpu.async_copy(
                gflat_hbm.at[gi_v.at[sl]], vg_v.at[sl], sem))
            copies.append(pltpu.async_copy(
                cs_hbm.at[il_v.at[sl]], vc_v.at[sl], sem))
        for j in range(dis_pw // _IDX_CHUNK):
            sl = pl.ds(j * _IDX_CHUNK, _IDX_CHUNK)
            copies.append(pltpu.async_copy(
                gflat_hbm.at[di_v.at[sl]], vd_v.at[sl], sem))
        for cp_ in copies:
            cp_.wait()
        lane = lax.iota(jnp.int32, _LANES)
        zero = jnp.zeros((_LANES,), jnp.float32)
        accg = zero
        accc = zero
        accd = zero
        for i in range(impl_pw // _LANES):
            sl = pl.ds(i * _LANES, _LANES)
            valid = (ib + i * _LANES) + lane < n_impl
            accg = accg + jnp.where(valid, vg_v[sl], 0.0)
            accc = accc + jnp.where(valid, vc_v[sl], 0.0)
        for i in range(dis_pw // _LANES):
            sl = pl.ds(i * _LANES, _LANES)
            valid = (db + i * _LANES) + lane < n_dis
            accd = accd + jnp.where(valid, vd_v[sl], 0.0)
        part_v[pl.ds(0, _LANES)] = accg
        part_v[pl.ds(_LANES, _LANES)] = accc
        part_v[pl.ds(2 * _LANES, _LANES)] = accd
        pltpu.sync_copy(part_v, out_hbm.at[wid])

    return sc_gather


def _ceil_to(n, m):
    return -(-n // m) * m


def kernel(input, target, impl_l, impl_r, dis_l, dis_r):
    b, c = input.shape

    n_impl = impl_l.shape[0]
    n_dis = dis_l.shape[0]
    impl_pw = _ceil_to(-(-n_impl // _NW), _IDX_CHUNK)
    dis_pw = _ceil_to(-(-n_dis // _NW), _IDX_CHUNK)
    il = jnp.pad(impl_l, (0, _NW * impl_pw - n_impl))
    ir = jnp.pad(impl_r, (0, _NW * impl_pw - n_impl))
    dl = jnp.pad(dis_l, (0, _NW * dis_pw - n_dis))
    dr = jnp.pad(dis_r, (0, _NW * dis_pw - n_dis))

    g, cs, bce = _tc_stage(input, target, row_tile=256)
    gflat = jnp.reshape(g, (c * c,))
    cs1 = jnp.reshape(cs, (c,))
    parts = _make_sc_gather(c, n_impl, n_dis, impl_pw, dis_pw)(
        gflat, cs1, il, ir, dl, dr)

    sums = jnp.sum(jnp.reshape(parts, (_NW, 3, _LANES)), axis=(0, 2))
    base_loss = bce[0, 0] / (b * c)
    implication_loss = (sums[1] - sums[0]) / b
    disjointness_loss = sums[2] / b
    loss = base_loss + 0.1 * implication_loss
    total = loss + 100.0 * disjointness_loss
    return (total, base_loss, implication_loss, disjointness_loss)


# speedup vs baseline: 1.3473x; 1.2792x over previous
import functools

import jax
import jax.numpy as jnp
from jax import lax
from jax.experimental import pallas as pl
from jax.experimental.pallas import tpu as pltpu
from jax.experimental.pallas import tpu_sc as plsc

_LANES = 16
_NC = 2
_NS = 16
_NW = _NC * _NS
_IDX_CHUNK = 128


def _tc_body(x_ref, t_ref, g_ref, cs_ref, bce_ref):
    i = pl.program_id(0)
    x = x_ref[...]
    t = t_ref[...]
    e = jnp.exp(-jnp.abs(x))
    bce = jnp.sum(jnp.maximum(x, 0.0) - x * t + jnp.log1p(e))
    s = 1.0 / (1.0 + e)
    pred = jnp.where(x >= 0, s, 1.0 - s)
    g = lax.dot_general(pred, pred, (((1,), (1,)), ((), ())),
                        preferred_element_type=jnp.float32,
                        precision=lax.Precision.DEFAULT)
    cs = jnp.sum(pred, axis=1, keepdims=True)

    @pl.when(i == 0)
    def _init():
        g_ref[...] = g
        cs_ref[...] = cs
        bce_ref[...] = jnp.reshape(bce, (1, 1))

    @pl.when(i > 0)
    def _acc():
        g_ref[...] += g
        cs_ref[...] += cs
        bce_ref[...] += jnp.reshape(bce, (1, 1))


def _tc_stage(xt, tt, col_tile):
    c, b = xt.shape
    return pl.pallas_call(
        _tc_body,
        grid=(b // col_tile,),
        in_specs=[pl.BlockSpec((c, col_tile), lambda i: (0, i)),
                  pl.BlockSpec((c, col_tile), lambda i: (0, i))],
        out_specs=[pl.BlockSpec((c, c), lambda i: (0, 0)),
                   pl.BlockSpec((c, 1), lambda i: (0, 0)),
                   pl.BlockSpec((1, 1), lambda i: (0, 0))],
        out_shape=[jax.ShapeDtypeStruct((c, c), jnp.float32),
                   jax.ShapeDtypeStruct((c, 1), jnp.float32),
                   jax.ShapeDtypeStruct((1, 1), jnp.float32)],
    )(xt, tt)


def _make_sc_gather(c, n_impl, n_dis, impl_pw, dis_pw):
    mesh = plsc.VectorSubcoreMesh(core_axis_name="c", subcore_axis_name="s")

    @functools.partial(
        pl.kernel,
        mesh=mesh,
        out_type=jax.ShapeDtypeStruct((_NW, 3 * _LANES), jnp.float32),
        scratch_types=[
            pltpu.VMEM((impl_pw,), jnp.int32),
            pltpu.VMEM((impl_pw,), jnp.int32),
            pltpu.VMEM((dis_pw,), jnp.int32),
            pltpu.VMEM((dis_pw,), jnp.int32),
            pltpu.VMEM((impl_pw,), jnp.int32),
            pltpu.VMEM((dis_pw,), jnp.int32),
            pltpu.VMEM((impl_pw,), jnp.float32),
            pltpu.VMEM((impl_pw,), jnp.float32),
            pltpu.VMEM((dis_pw,), jnp.float32),
            pltpu.VMEM((3 * _LANES,), jnp.float32),
            pltpu.SemaphoreType.DMA,
        ],
    )
    def sc_gather(gflat_hbm, cs_hbm, il_hbm, ir_hbm, dl_hbm, dr_hbm, out_hbm,
                  il_v, ir_v, dl_v, dr_v, gi_v, di_v,
                  vg_v, vc_v, vd_v, part_v, sem):
        wid = lax.axis_index("s") * _NC + lax.axis_index("c")
        ib = wid * impl_pw
        db = wid * dis_pw
        pltpu.sync_copy(il_hbm.at[pl.ds(ib, impl_pw)], il_v)
        pltpu.sync_copy(ir_hbm.at[pl.ds(ib, impl_pw)], ir_v)
        pltpu.sync_copy(dl_hbm.at[pl.ds(db, dis_pw)], dl_v)
        pltpu.sync_copy(dr_hbm.at[pl.ds(db, dis_pw)], dr_v)
        for i in range(impl_pw // _LANES):
            sl = pl.ds(i * _LANES, _LANES)
            gi_v[sl] = il_v[sl] * c + ir_v[sl]
        for i in range(dis_pw // _LANES):
            sl = pl.ds(i * _LANES, _LANES)
            di_v[sl] = dl_v[sl] * c + dr_v[sl]
        copies = []
        for j in range(impl_pw // _IDX_CHUNK):
            sl = pl.ds(j * _IDX_CHUNK, _IDX_CHUNK)
            copies.append(pltpu.async_copy(
                gflat_hbm.at[gi_v.at[sl]], vg_v.at[sl], sem))
            copies.append(pltpu.async_copy(
                cs_hbm.at[il_v.at[sl]], vc_v.at[sl], sem))
        for j in range(dis_pw // _IDX_CHUNK):
            sl = pl.ds(j * _IDX_CHUNK, _IDX_CHUNK)
            copies.append(pltpu.async_copy(
                gflat_hbm.at[di_v.at[sl]], vd_v.at[sl], sem))
        for cp_ in copies:
            cp_.wait()
        lane = lax.iota(jnp.int32, _LANES)
        zero = jnp.zeros((_LANES,), jnp.float32)
        accg = zero
        accc = zero
        accd = zero
        for i in range(impl_pw // _LANES):
            sl = pl.ds(i * _LANES, _LANES)
            valid = (ib + i * _LANES) + lane < n_impl
            accg = accg + jnp.where(valid, vg_v[sl], 0.0)
            accc = accc + jnp.where(valid, vc_v[sl], 0.0)
        for i in range(dis_pw // _LANES):
            sl = pl.ds(i * _LANES, _LANES)
            valid = (db + i * _LANES) + lane < n_dis
            accd = accd + jnp.where(valid, vd_v[sl], 0.0)
        part_v[pl.ds(0, _LANES)] = accg
        part_v[pl.ds(_LANES, _LANES)] = accc
        part_v[pl.ds(2 * _LANES, _LANES)] = accd
        pltpu.sync_copy(part_v, out_hbm.at[wid])

    return sc_gather


def _ceil_to(n, m):
    return -(-n // m) * m


def kernel(input, target, impl_l, impl_r, dis_l, dis_r):
    b, c = input.shape

    n_impl = impl_l.shape[0]
    n_dis = dis_l.shape[0]
    impl_pw = _ceil_to(-(-n_impl // _NW), _IDX_CHUNK)
    dis_pw = _ceil_to(-(-n_dis // _NW), _IDX_CHUNK)
    il = jnp.pad(impl_l, (0, _NW * impl_pw - n_impl))
    ir = jnp.pad(impl_r, (0, _NW * impl_pw - n_impl))
    dl = jnp.pad(dis_l, (0, _NW * dis_pw - n_dis))
    dr = jnp.pad(dis_r, (0, _NW * dis_pw - n_dis))

    g, cs, bce = _tc_stage(input.T, target.T, col_tile=256)
    gflat = jnp.reshape(g, (c * c,))
    cs1 = jnp.reshape(cs, (c,))
    parts = _make_sc_gather(c, n_impl, n_dis, impl_pw, dis_pw)(
        gflat, cs1, il, ir, dl, dr)

    sums = jnp.sum(jnp.reshape(parts, (_NW, 3, _LANES)), axis=(0, 2))
    base_loss = bce[0, 0] / (b * c)
    implication_loss = (sums[1] - sums[0]) / b
    disjointness_loss = sums[2] / b
    loss = base_loss + 0.1 * implication_loss
    total = loss + 100.0 * disjointness_loss
    return (total, base_loss, implication_loss, disjointness_loss)


# speedup vs baseline: 1.4007x; 1.0396x over previous
import functools

import jax
import jax.numpy as jnp
from jax import lax
from jax.experimental import pallas as pl
from jax.experimental.pallas import tpu as pltpu
from jax.experimental.pallas import tpu_sc as plsc

_LANES = 16
_NC = 2
_NS = 16
_NW = _NC * _NS
_IDX_CHUNK = 128


def _tc_body(x_ref, t_ref, g_ref, cs_ref, bce_ref):
    i = pl.program_id(0)
    x = x_ref[...]
    t = t_ref[...]
    e = jnp.exp(-jnp.abs(x))
    bce = jnp.sum(jnp.maximum(x, 0.0) - x * t + jnp.log1p(e))
    s = 1.0 / (1.0 + e)
    pred = jnp.where(x >= 0, s, 1.0 - s)
    g = lax.dot_general(pred, pred, (((1,), (1,)), ((), ())),
                        preferred_element_type=jnp.float32,
                        precision=lax.Precision.DEFAULT)
    cs = jnp.sum(pred, axis=1, keepdims=True)

    @pl.when(i == 0)
    def _init():
        g_ref[...] = g
        cs_ref[...] = cs
        bce_ref[...] = jnp.reshape(bce, (1, 1))

    @pl.when(i > 0)
    def _acc():
        g_ref[...] += g
        cs_ref[...] += cs
        bce_ref[...] += jnp.reshape(bce, (1, 1))


def _tc_stage(xt, tt, col_tile):
    c, b = xt.shape
    return pl.pallas_call(
        _tc_body,
        grid=(b // col_tile,),
        in_specs=[pl.BlockSpec((c, col_tile), lambda i: (0, i)),
                  pl.BlockSpec((c, col_tile), lambda i: (0, i))],
        out_specs=[pl.BlockSpec((c, c), lambda i: (0, 0)),
                   pl.BlockSpec((c, 1), lambda i: (0, 0)),
                   pl.BlockSpec((1, 1), lambda i: (0, 0))],
        out_shape=[jax.ShapeDtypeStruct((c, c), jnp.float32),
                   jax.ShapeDtypeStruct((c, 1), jnp.float32),
                   jax.ShapeDtypeStruct((1, 1), jnp.float32)],
    )(xt, tt)


def _make_sc_gather(c, n_impl, n_dis, impl_pw, dis_pw):
    mesh = plsc.VectorSubcoreMesh(core_axis_name="c", subcore_axis_name="s")

    @functools.partial(
        pl.kernel,
        mesh=mesh,
        out_type=jax.ShapeDtypeStruct((_NW, 3 * _LANES), jnp.float32),
        scratch_types=[
            pltpu.VMEM((impl_pw,), jnp.int32),
            pltpu.VMEM((impl_pw,), jnp.int32),
            pltpu.VMEM((dis_pw,), jnp.int32),
            pltpu.VMEM((dis_pw,), jnp.int32),
            pltpu.VMEM((impl_pw,), jnp.int32),
            pltpu.VMEM((dis_pw,), jnp.int32),
            pltpu.VMEM((impl_pw,), jnp.float32),
            pltpu.VMEM((impl_pw,), jnp.float32),
            pltpu.VMEM((dis_pw,), jnp.float32),
            pltpu.VMEM((3 * _LANES,), jnp.float32),
            pltpu.SemaphoreType.DMA,
        ],
    )
    def sc_gather(gflat_hbm, cs_hbm, il_hbm, ir_hbm, dl_hbm, dr_hbm, out_hbm,
                  il_v, ir_v, dl_v, dr_v, gi_v, di_v,
                  vg_v, vc_v, vd_v, part_v, sem):
        wid = lax.axis_index("s") * _NC + lax.axis_index("c")
        ib = wid * impl_pw
        db = wid * dis_pw
        pltpu.sync_copy(il_hbm.at[pl.ds(ib, impl_pw)], il_v)
        pltpu.sync_copy(ir_hbm.at[pl.ds(ib, impl_pw)], ir_v)
        pltpu.sync_copy(dl_hbm.at[pl.ds(db, dis_pw)], dl_v)
        pltpu.sync_copy(dr_hbm.at[pl.ds(db, dis_pw)], dr_v)
        for i in range(impl_pw // _LANES):
            sl = pl.ds(i * _LANES, _LANES)
            gi_v[sl] = il_v[sl] * c + ir_v[sl]
        for i in range(dis_pw // _LANES):
            sl = pl.ds(i * _LANES, _LANES)
            di_v[sl] = dl_v[sl] * c + dr_v[sl]
        copies = []
        for j in range(impl_pw // _IDX_CHUNK):
            sl = pl.ds(j * _IDX_CHUNK, _IDX_CHUNK)
            copies.append(pltpu.async_copy(
                gflat_hbm.at[gi_v.at[sl]], vg_v.at[sl], sem))
            copies.append(pltpu.async_copy(
                cs_hbm.at[il_v.at[sl]], vc_v.at[sl], sem))
        for j in range(dis_pw // _IDX_CHUNK):
            sl = pl.ds(j * _IDX_CHUNK, _IDX_CHUNK)
            copies.append(pltpu.async_copy(
                gflat_hbm.at[di_v.at[sl]], vd_v.at[sl], sem))
        for cp_ in copies:
            cp_.wait()
        lane = lax.iota(jnp.int32, _LANES)
        zero = jnp.zeros((_LANES,), jnp.float32)
        accg = zero
        accc = zero
        accd = zero
        for i in range(impl_pw // _LANES):
            sl = pl.ds(i * _LANES, _LANES)
            valid = (ib + i * _LANES) + lane < n_impl
            accg = accg + jnp.where(valid, vg_v[sl], 0.0)
            accc = accc + jnp.where(valid, vc_v[sl], 0.0)
        for i in range(dis_pw // _LANES):
            sl = pl.ds(i * _LANES, _LANES)
            valid = (db + i * _LANES) + lane < n_dis
            accd = accd + jnp.where(valid, vd_v[sl], 0.0)
        part_v[pl.ds(0, _LANES)] = accg
        part_v[pl.ds(_LANES, _LANES)] = accc
        part_v[pl.ds(2 * _LANES, _LANES)] = accd
        pltpu.sync_copy(part_v, out_hbm.at[wid])

    return sc_gather


def _ceil_to(n, m):
    return -(-n // m) * m


def kernel(input, target, impl_l, impl_r, dis_l, dis_r):
    b, c = input.shape

    n_impl = impl_l.shape[0]
    n_dis = dis_l.shape[0]
    impl_pw = _ceil_to(-(-n_impl // _NW), _IDX_CHUNK)
    dis_pw = _ceil_to(-(-n_dis // _NW), _IDX_CHUNK)
    il = jnp.pad(impl_l, (0, _NW * impl_pw - n_impl))
    ir = jnp.pad(impl_r, (0, _NW * impl_pw - n_impl))
    dl = jnp.pad(dis_l, (0, _NW * dis_pw - n_dis))
    dr = jnp.pad(dis_r, (0, _NW * dis_pw - n_dis))

    g, cs, bce = _tc_stage(input.T, target.T, col_tile=512)
    gflat = jnp.reshape(g, (c * c,))
    cs1 = jnp.reshape(cs, (c,))
    parts = _make_sc_gather(c, n_impl, n_dis, impl_pw, dis_pw)(
        gflat, cs1, il, ir, dl, dr)

    sums = jnp.sum(jnp.reshape(parts, (_NW, 3, _LANES)), axis=(0, 2))
    base_loss = bce[0, 0] / (b * c)
    implication_loss = (sums[1] - sums[0]) / b
    disjointness_loss = sums[2] / b
    loss = base_loss + 0.1 * implication_loss
    total = loss + 100.0 * disjointness_loss
    return (total, base_loss, implication_loss, disjointness_loss)


# speedup vs baseline: 1.5205x; 1.0856x over previous
import functools

import jax
import jax.numpy as jnp
from jax import lax
from jax.experimental import pallas as pl
from jax.experimental.pallas import tpu as pltpu
from jax.experimental.pallas import tpu_sc as plsc

_LANES = 16
_NC = 2
_NS = 16
_NW = _NC * _NS
_IDX_CHUNK = 128


def _tc_body(x_ref, t_ref, g_ref, cs_ref, bce_ref, *, c, cpad):
    i = pl.program_id(0)
    x = x_ref[...]
    t = t_ref[...]
    e = jnp.exp(-jnp.abs(x))
    bce = jnp.sum(jnp.maximum(x, 0.0) - x * t + jnp.log1p(e))
    s = 1.0 / (1.0 + e)
    pred = jnp.where(x >= 0, s, 1.0 - s)
    g = lax.dot_general(pred, pred, (((1,), (1,)), ((), ())),
                        preferred_element_type=jnp.float32,
                        precision=lax.Precision.DEFAULT)
    cs = jnp.sum(pred, axis=1, keepdims=True)
    nslab = g_ref.shape[0]

    @pl.when(i == 0)
    def _init():
        for j in range(nslab):
            w = min(128, c - j * 128)
            g_ref[j, pl.ds(0, c), pl.ds(0, w)] = g[:, j * 128:j * 128 + w]
        cs_ref[...] = cs
        bce_ref[...] = jnp.reshape(bce, (1, 1))

    @pl.when(i > 0)
    def _acc():
        for j in range(nslab):
            w = min(128, c - j * 128)
            g_ref[j, pl.ds(0, c), pl.ds(0, w)] += g[:, j * 128:j * 128 + w]
        cs_ref[...] += cs
        bce_ref[...] += jnp.reshape(bce, (1, 1))


def _tc_stage(xt, tt, col_tile):
    c, b = xt.shape
    cpad = _ceil_to(c, 8)
    nslab = -(-c // 128)
    return pl.pallas_call(
        functools.partial(_tc_body, c=c, cpad=cpad),
        grid=(b // col_tile,),
        in_specs=[pl.BlockSpec((c, col_tile), lambda i: (0, i)),
                  pl.BlockSpec((c, col_tile), lambda i: (0, i))],
        out_specs=[pl.BlockSpec((nslab, cpad, 128), lambda i: (0, 0, 0)),
                   pl.BlockSpec((c, 1), lambda i: (0, 0)),
                   pl.BlockSpec((1, 1), lambda i: (0, 0))],
        out_shape=[jax.ShapeDtypeStruct((nslab, cpad, 128), jnp.float32),
                   jax.ShapeDtypeStruct((c, 1), jnp.float32),
                   jax.ShapeDtypeStruct((1, 1), jnp.float32)],
    )(xt, tt)


def _make_sc_gather(slab_stride, n_impl, n_dis, impl_pw, dis_pw):
    mesh = plsc.VectorSubcoreMesh(core_axis_name="c", subcore_axis_name="s")

    @functools.partial(
        pl.kernel,
        mesh=mesh,
        out_type=jax.ShapeDtypeStruct((_NW, 3 * _LANES), jnp.float32),
        scratch_types=[
            pltpu.VMEM((impl_pw,), jnp.int32),
            pltpu.VMEM((impl_pw,), jnp.int32),
            pltpu.VMEM((dis_pw,), jnp.int32),
            pltpu.VMEM((dis_pw,), jnp.int32),
            pltpu.VMEM((impl_pw,), jnp.int32),
            pltpu.VMEM((dis_pw,), jnp.int32),
            pltpu.VMEM((impl_pw,), jnp.float32),
            pltpu.VMEM((impl_pw,), jnp.float32),
            pltpu.VMEM((dis_pw,), jnp.float32),
            pltpu.VMEM((3 * _LANES,), jnp.float32),
            pltpu.SemaphoreType.DMA,
        ],
    )
    def sc_gather(gflat_hbm, cs_hbm, il_hbm, ir_hbm, dl_hbm, dr_hbm, out_hbm,
                  il_v, ir_v, dl_v, dr_v, gi_v, di_v,
                  vg_v, vc_v, vd_v, part_v, sem):
        wid = lax.axis_index("s") * _NC + lax.axis_index("c")
        ib = wid * impl_pw
        db = wid * dis_pw
        pltpu.sync_copy(il_hbm.at[pl.ds(ib, impl_pw)], il_v)
        pltpu.sync_copy(ir_hbm.at[pl.ds(ib, impl_pw)], ir_v)
        pltpu.sync_copy(dl_hbm.at[pl.ds(db, dis_pw)], dl_v)
        pltpu.sync_copy(dr_hbm.at[pl.ds(db, dis_pw)], dr_v)
        for i in range(impl_pw // _LANES):
            sl = pl.ds(i * _LANES, _LANES)
            r = ir_v[sl]
            gi_v[sl] = (r >> 7) * slab_stride + (il_v[sl] << 7) + (r & 127)
        for i in range(dis_pw // _LANES):
            sl = pl.ds(i * _LANES, _LANES)
            r = dr_v[sl]
            di_v[sl] = (r >> 7) * slab_stride + (dl_v[sl] << 7) + (r & 127)
        copies = []
        for j in range(impl_pw // _IDX_CHUNK):
            sl = pl.ds(j * _IDX_CHUNK, _IDX_CHUNK)
            copies.append(pltpu.async_copy(
                gflat_hbm.at[gi_v.at[sl]], vg_v.at[sl], sem))
            copies.append(pltpu.async_copy(
                cs_hbm.at[il_v.at[sl]], vc_v.at[sl], sem))
        for j in range(dis_pw // _IDX_CHUNK):
            sl = pl.ds(j * _IDX_CHUNK, _IDX_CHUNK)
            copies.append(pltpu.async_copy(
                gflat_hbm.at[di_v.at[sl]], vd_v.at[sl], sem))
        for cp_ in copies:
            cp_.wait()
        lane = lax.iota(jnp.int32, _LANES)
        zero = jnp.zeros((_LANES,), jnp.float32)
        accg = zero
        accc = zero
        accd = zero
        for i in range(impl_pw // _LANES):
            sl = pl.ds(i * _LANES, _LANES)
            valid = (ib + i * _LANES) + lane < n_impl
            accg = accg + jnp.where(valid, vg_v[sl], 0.0)
            accc = accc + jnp.where(valid, vc_v[sl], 0.0)
        for i in range(dis_pw // _LANES):
            sl = pl.ds(i * _LANES, _LANES)
            valid = (db + i * _LANES) + lane < n_dis
            accd = accd + jnp.where(valid, vd_v[sl], 0.0)
        part_v[pl.ds(0, _LANES)] = accg
        part_v[pl.ds(_LANES, _LANES)] = accc
        part_v[pl.ds(2 * _LANES, _LANES)] = accd
        pltpu.sync_copy(part_v, out_hbm.at[wid])

    return sc_gather


def _ceil_to(n, m):
    return -(-n // m) * m


def kernel(input, target, impl_l, impl_r, dis_l, dis_r):
    b, c = input.shape

    n_impl = impl_l.shape[0]
    n_dis = dis_l.shape[0]
    impl_pw = _ceil_to(-(-n_impl // _NW), _IDX_CHUNK)
    dis_pw = _ceil_to(-(-n_dis // _NW), _IDX_CHUNK)
    il = jnp.pad(impl_l, (0, _NW * impl_pw - n_impl))
    ir = jnp.pad(impl_r, (0, _NW * impl_pw - n_impl))
    dl = jnp.pad(dis_l, (0, _NW * dis_pw - n_dis))
    dr = jnp.pad(dis_r, (0, _NW * dis_pw - n_dis))

    g, cs, bce = _tc_stage(input.T, target.T, col_tile=512)
    nslab, cpad, _ = g.shape
    gflat = jnp.reshape(g, (nslab * cpad * 128,))
    cs1 = jnp.reshape(cs, (c,))
    parts = _make_sc_gather(cpad * 128, n_impl, n_dis, impl_pw, dis_pw)(
        gflat, cs1, il, ir, dl, dr)

    sums = jnp.sum(jnp.reshape(parts, (_NW, 3, _LANES)), axis=(0, 2))
    base_loss = bce[0, 0] / (b * c)
    implication_loss = (sums[1] - sums[0]) / b
    disjointness_loss = sums[2] / b
    loss = base_loss + 0.1 * implication_loss
    total = loss + 100.0 * disjointness_loss
    return (total, base_loss, implication_loss, disjointness_loss)


# speedup vs baseline: 2.1395x; 1.4071x over previous
import functools

import jax
import jax.numpy as jnp
from jax import lax
from jax.experimental import pallas as pl
from jax.experimental.pallas import tpu as pltpu
from jax.experimental.pallas import tpu_sc as plsc

_LANES = 16
_NC = 2
_NS = 16
_NW = _NC * _NS
_IDX_CHUNK = 128


def _tc_body(x_ref, t_ref, g_ref, bce_ref, *, c, cpad):
    i = pl.program_id(0)
    x = x_ref[...]
    t = t_ref[...]
    e = jnp.exp(-jnp.abs(x))
    bce = jnp.sum(jnp.maximum(x, 0.0) - x * t + jnp.log1p(e))
    s = 1.0 / (1.0 + e)
    pred = jnp.where(x >= 0, s, 1.0 - s)
    g = lax.dot_general(pred, pred, (((1,), (1,)), ((), ())),
                        preferred_element_type=jnp.float32,
                        precision=lax.Precision.DEFAULT)
    cs = jnp.sum(pred, axis=1, keepdims=True)
    nslab = g_ref.shape[0] - 1

    @pl.when(i == 0)
    def _init():
        for j in range(nslab):
            w = min(128, c - j * 128)
            g_ref[j, pl.ds(0, c), pl.ds(0, w)] = g[:, j * 128:j * 128 + w]
        g_ref[nslab, pl.ds(0, c), pl.ds(0, 1)] = cs
        bce_ref[...] = jnp.reshape(bce, (1, 1))

    @pl.when(i > 0)
    def _acc():
        for j in range(nslab):
            w = min(128, c - j * 128)
            g_ref[j, pl.ds(0, c), pl.ds(0, w)] += g[:, j * 128:j * 128 + w]
        g_ref[nslab, pl.ds(0, c), pl.ds(0, 1)] += cs
        bce_ref[...] += jnp.reshape(bce, (1, 1))


def _tc_stage(xt, tt, col_tile):
    c, b = xt.shape
    cpad = _ceil_to(c, 8)
    nslab = -(-c // 128)
    return pl.pallas_call(
        functools.partial(_tc_body, c=c, cpad=cpad),
        grid=(b // col_tile,),
        in_specs=[pl.BlockSpec((c, col_tile), lambda i: (0, i)),
                  pl.BlockSpec((c, col_tile), lambda i: (0, i))],
        out_specs=[pl.BlockSpec((nslab + 1, cpad, 128), lambda i: (0, 0, 0)),
                   pl.BlockSpec((1, 1), lambda i: (0, 0))],
        out_shape=[jax.ShapeDtypeStruct((nslab + 1, cpad, 128), jnp.float32),
                   jax.ShapeDtypeStruct((1, 1), jnp.float32)],
    )(xt, tt)


def _make_sc_gather(slab_stride, cs_off, n_impl, n_dis, impl_pw, dis_pw):
    mesh = plsc.VectorSubcoreMesh(core_axis_name="c", subcore_axis_name="s")

    @functools.partial(
        pl.kernel,
        mesh=mesh,
        out_type=jax.ShapeDtypeStruct((_NW, 3 * _LANES), jnp.float32),
        scratch_types=[
            pltpu.VMEM((impl_pw,), jnp.int32),
            pltpu.VMEM((impl_pw,), jnp.int32),
            pltpu.VMEM((dis_pw,), jnp.int32),
            pltpu.VMEM((dis_pw,), jnp.int32),
            pltpu.VMEM((impl_pw,), jnp.int32),
            pltpu.VMEM((impl_pw,), jnp.int32),
            pltpu.VMEM((dis_pw,), jnp.int32),
            pltpu.VMEM((impl_pw,), jnp.float32),
            pltpu.VMEM((impl_pw,), jnp.float32),
            pltpu.VMEM((dis_pw,), jnp.float32),
            pltpu.VMEM((3 * _LANES,), jnp.float32),
            pltpu.SemaphoreType.DMA,
        ],
    )
    def sc_gather(gflat_hbm, il_hbm, ir_hbm, dl_hbm, dr_hbm, out_hbm,
                  il_v, ir_v, dl_v, dr_v, gi_v, ci_v, di_v,
                  vg_v, vc_v, vd_v, part_v, sem):
        wid = lax.axis_index("s") * _NC + lax.axis_index("c")
        ib = wid * impl_pw
        db = wid * dis_pw
        ibc = jnp.minimum(ib, (n_impl - impl_pw) & ~7)
        dbc = jnp.minimum(db, (n_dis - dis_pw) & ~7)
        loads = [
            pltpu.async_copy(il_hbm.at[pl.ds(ibc, impl_pw)], il_v, sem),
            pltpu.async_copy(ir_hbm.at[pl.ds(ibc, impl_pw)], ir_v, sem),
            pltpu.async_copy(dl_hbm.at[pl.ds(dbc, dis_pw)], dl_v, sem),
            pltpu.async_copy(dr_hbm.at[pl.ds(dbc, dis_pw)], dr_v, sem),
        ]
        for ld in loads:
            ld.wait()
        for i in range(impl_pw // _LANES):
            sl = pl.ds(i * _LANES, _LANES)
            l = il_v[sl]
            r = ir_v[sl]
            gi_v[sl] = (r >> 7) * slab_stride + (l << 7) + (r & 127)
            ci_v[sl] = (l << 7) + cs_off
        for i in range(dis_pw // _LANES):
            sl = pl.ds(i * _LANES, _LANES)
            r = dr_v[sl]
            di_v[sl] = (r >> 7) * slab_stride + (dl_v[sl] << 7) + (r & 127)
        copies = []
        for j in range(impl_pw // _IDX_CHUNK):
            sl = pl.ds(j * _IDX_CHUNK, _IDX_CHUNK)
            copies.append(pltpu.async_copy(
                gflat_hbm.at[gi_v.at[sl]], vg_v.at[sl], sem))
            copies.append(pltpu.async_copy(
                gflat_hbm.at[ci_v.at[sl]], vc_v.at[sl], sem))
        for j in range(dis_pw // _IDX_CHUNK):
            sl = pl.ds(j * _IDX_CHUNK, _IDX_CHUNK)
            copies.append(pltpu.async_copy(
                gflat_hbm.at[di_v.at[sl]], vd_v.at[sl], sem))
        for cp_ in copies:
            cp_.wait()
        lane = lax.iota(jnp.int32, _LANES)
        zero = jnp.zeros((_LANES,), jnp.float32)
        accg = zero
        accc = zero
        accd = zero
        for i in range(impl_pw // _LANES):
            sl = pl.ds(i * _LANES, _LANES)
            pos = ibc + i * _LANES + lane
            valid = (pos >= ib) & (pos < n_impl)
            accg = accg + jnp.where(valid, vg_v[sl], 0.0)
            accc = accc + jnp.where(valid, vc_v[sl], 0.0)
        for i in range(dis_pw // _LANES):
            sl = pl.ds(i * _LANES, _LANES)
            pos = dbc + i * _LANES + lane
            valid = (pos >= db) & (pos < n_dis)
            accd = accd + jnp.where(valid, vd_v[sl], 0.0)
        part_v[pl.ds(0, _LANES)] = accg
        part_v[pl.ds(_LANES, _LANES)] = accc
        part_v[pl.ds(2 * _LANES, _LANES)] = accd
        pltpu.sync_copy(part_v, out_hbm.at[wid])

    return sc_gather


def _ceil_to(n, m):
    return -(-n // m) * m


def kernel(input, target, impl_l, impl_r, dis_l, dis_r):
    b, c = input.shape

    n_impl = impl_l.shape[0]
    n_dis = dis_l.shape[0]
    impl_pw = _ceil_to(-(-n_impl // _NW), _IDX_CHUNK)
    dis_pw = _ceil_to(-(-n_dis // _NW), _IDX_CHUNK)

    g, bce = _tc_stage(input.T, target.T, col_tile=512)
    nslab1, cpad, _ = g.shape
    gflat = jnp.reshape(g, (nslab1 * cpad * 128,))
    cs_off = (nslab1 - 1) * cpad * 128
    parts = _make_sc_gather(cpad * 128, cs_off, n_impl, n_dis,
                            impl_pw, dis_pw)(
        gflat, impl_l, impl_r, dis_l, dis_r)

    sums = jnp.sum(jnp.reshape(parts, (_NW, 3, _LANES)), axis=(0, 2))
    base_loss = bce[0, 0] / (b * c)
    implication_loss = (sums[1] - sums[0]) / b
    disjointness_loss = sums[2] / b
    loss = base_loss + 0.1 * implication_loss
    total = loss + 100.0 * disjointness_loss
    return (total, base_loss, implication_loss, disjointness_loss)
